# baseline (speedup 1.0000x reference)
import jax
import jax.numpy as jnp
import numpy as np
from jax.experimental import pallas as pl
from jax.experimental.pallas import tpu as pltpu

_ROT_A = (13, 15, 26, 6)
_ROT_B = (17, 29, 16, 24)
_K0 = 0
_K1 = 42
_KS = (np.uint32(_K0), np.uint32(_K1), np.uint32(_K0 ^ _K1 ^ 0x1BD11BDA))
_TINY = np.float32(np.finfo(np.float32).tiny)

_BLOCK_ROWS = 8


def _rotl(x, r):
    return (x << np.uint32(r)) | (x >> np.uint32(32 - r))


def _threefry2x32(x0, x1):
    x0 = x0 + _KS[0]
    x1 = x1 + _KS[1]
    rots = (_ROT_A, _ROT_B)
    for i in range(5):
        for r in rots[i % 2]:
            x0 = x0 + x1
            x1 = _rotl(x1, r)
            x1 = x0 ^ x1
        x0 = x0 + _KS[(i + 1) % 3]
        x1 = x1 + _KS[(i + 2) % 3] + np.uint32(i + 1)
    return x0, x1


def _gumbel_from_flat(flat_u32):
    zeros = jnp.zeros_like(flat_u32)
    b0, b1 = _threefry2x32(zeros, flat_u32)
    bits = b0 ^ b1
    fb = (bits >> np.uint32(9)) | np.uint32(0x3F800000)
    f = jax.lax.bitcast_convert_type(fb, jnp.float32) - np.float32(1.0)
    u = jnp.maximum(_TINY, f * (np.float32(1.0) - _TINY) + _TINY)
    return -jnp.log(-jnp.log(u))


def _body(x_ref, o_ref, *, cols):
    i = pl.program_id(0)
    shape = x_ref.shape
    base = (i * (shape[0] * cols)).astype(jnp.uint32)
    row = jax.lax.broadcasted_iota(jnp.uint32, shape, 0)
    col = jax.lax.broadcasted_iota(jnp.uint32, shape, 1)
    flat = base + row * np.uint32(cols) + col
    g = _gumbel_from_flat(flat)
    z = x_ref[...] + g
    m = jnp.max(z, axis=1, keepdims=True)
    e = jnp.exp(z - m)
    s = jnp.sum(e, axis=1, keepdims=True)
    o_ref[...] = e * (np.float32(1.0) / s)


def kernel(logits):
    rows, cols = logits.shape
    block = _BLOCK_ROWS if rows % _BLOCK_ROWS == 0 else 1
    grid = rows // block
    import functools
    return pl.pallas_call(
        functools.partial(_body, cols=cols),
        grid=(grid,),
        in_specs=[pl.BlockSpec((block, cols), lambda i: (i, 0))],
        out_specs=pl.BlockSpec((block, cols), lambda i: (i, 0)),
        out_shape=jax.ShapeDtypeStruct((rows, cols), logits.dtype),
        compiler_params=pltpu.CompilerParams(
            dimension_semantics=("arbitrary",),
        ),
    )(logits)


# speedup vs baseline: 1.0367x; 1.0367x over previous
import functools

import jax
import jax.numpy as jnp
import numpy as np
from jax.experimental import pallas as pl
from jax.experimental.pallas import tpu as pltpu

_ROT_A = (13, 15, 26, 6)
_ROT_B = (17, 29, 16, 24)
_K0 = 0
_K1 = 42
_KS = (np.uint32(_K0), np.uint32(_K1), np.uint32(_K0 ^ _K1 ^ 0x1BD11BDA))
_TINY = np.float32(np.finfo(np.float32).tiny)

_BLOCK_ROWS = 8
_CHUNK = 2048


def _rotl(x, r):
    return (x << np.uint32(r)) | (x >> np.uint32(32 - r))


def _threefry2x32(x0, x1):
    x0 = x0 + _KS[0]
    x1 = x1 + _KS[1]
    rots = (_ROT_A, _ROT_B)
    for i in range(5):
        for r in rots[i % 2]:
            x0 = x0 + x1
            x1 = _rotl(x1, r)
            x1 = x0 ^ x1
        x0 = x0 + _KS[(i + 1) % 3]
        x1 = x1 + _KS[(i + 2) % 3] + np.uint32(i + 1)
    return x0, x1


def _gumbel_from_flat(flat_u32):
    zeros = jnp.zeros_like(flat_u32)
    b0, b1 = _threefry2x32(zeros, flat_u32)
    bits = b0 ^ b1
    fb = (bits >> np.uint32(9)) | np.uint32(0x3F800000)
    f = jax.lax.bitcast_convert_type(fb, jnp.float32) - np.float32(1.0)
    u = jnp.maximum(_TINY, f * (np.float32(1.0) - _TINY) + _TINY)
    return -jnp.log(-jnp.log(u))


def _body(x_ref, o_ref, *, cols):
    rows = x_ref.shape[0]
    base = (pl.program_id(0) * (rows * cols)).astype(jnp.uint32)

    nfull, rem = divmod(cols, _CHUNK)
    row_term = jax.lax.broadcasted_iota(jnp.uint32, (rows, _CHUNK), 0) * np.uint32(cols)
    col_term = jax.lax.broadcasted_iota(jnp.uint32, (rows, _CHUNK), 1)
    inv_full = row_term + col_term

    def z_chunk(cs, width, inv):
        flat = inv + (base + jnp.uint32(cs))
        g = _gumbel_from_flat(flat)
        return x_ref[:, pl.ds(cs, width)] + g

    def pass_a(j, m):
        z = z_chunk(j * _CHUNK, _CHUNK, inv_full)
        o_ref[:, pl.ds(j * _CHUNK, _CHUNK)] = z
        return jnp.maximum(m, jnp.max(z, axis=1, keepdims=True))

    m = jnp.full((rows, 1), -jnp.inf, dtype=jnp.float32)
    if nfull:
        m = jax.lax.fori_loop(0, nfull, pass_a, m)
    if rem:
        z = z_chunk(nfull * _CHUNK, rem, inv_full[:, :rem])
        o_ref[:, pl.ds(nfull * _CHUNK, rem)] = z
        m = jnp.maximum(m, jnp.max(z, axis=1, keepdims=True))

    def pass_b(j, s):
        e = jnp.exp(o_ref[:, pl.ds(j * _CHUNK, _CHUNK)] - m)
        o_ref[:, pl.ds(j * _CHUNK, _CHUNK)] = e
        return s + jnp.sum(e, axis=1, keepdims=True)

    s = jnp.zeros((rows, 1), dtype=jnp.float32)
    if nfull:
        s = jax.lax.fori_loop(0, nfull, pass_b, s)
    if rem:
        e = jnp.exp(o_ref[:, pl.ds(nfull * _CHUNK, rem)] - m)
        o_ref[:, pl.ds(nfull * _CHUNK, rem)] = e
        s = s + jnp.sum(e, axis=1, keepdims=True)

    inv_s = np.float32(1.0) / s

    def pass_c(j, carry):
        o_ref[:, pl.ds(j * _CHUNK, _CHUNK)] *= inv_s
        return carry

    if nfull:
        jax.lax.fori_loop(0, nfull, pass_c, 0)
    if rem:
        o_ref[:, pl.ds(nfull * _CHUNK, rem)] *= inv_s


def kernel(logits):
    rows, cols = logits.shape
    block = _BLOCK_ROWS if rows % _BLOCK_ROWS == 0 else 1
    grid = rows // block
    return pl.pallas_call(
        functools.partial(_body, cols=cols),
        grid=(grid,),
        in_specs=[pl.BlockSpec((block, cols), lambda i: (i, 0))],
        out_specs=pl.BlockSpec((block, cols), lambda i: (i, 0)),
        out_shape=jax.ShapeDtypeStruct((rows, cols), logits.dtype),
        compiler_params=pltpu.CompilerParams(
            dimension_semantics=("arbitrary",),
        ),
    )(logits)
